# Initial kernel scaffold; baseline (speedup 1.0000x reference)
import jax
import jax.numpy as jnp
from jax.experimental import pallas as pl


def kernel(x_t, x_t_dt, edge_index, dt, W_node, b_node, W_edge, b_edge, W_q, b_q, W_k, b_k, W_v, b_v, W_skip, b_skip, W_cls, b_cls):
    raise NotImplementedError("write your pallas kernel here")


# speedup vs baseline: 47.5590x; 47.5590x over previous
import functools

import jax
import jax.numpy as jnp
from jax import lax
from jax.experimental import pallas as pl
from jax.experimental.pallas import tpu as pltpu
from jax.experimental.pallas import tpu_sc as plsc

N = 100000
E = 1600000
H = 4
D = 32
C = D // H
HALF = 16
SCALE = 1.0 / (C ** 0.5)

BLK = 4000

NSC = 2
NSUB = 16
LANES = 16

ECHUNK = 128
NCHUNKS = E // ECHUNK
ROWS_T = 6256
N_PAD = ROWS_T * NSUB

OCHUNK = 2000
EPW = E // (NSC * NSUB)


def _node_body(dt_ref, mp_ref, x_ref, xdt_ref, wn_ref, bn_ref, we_ref, be_ref,
               wq_ref, bq_ref, wk_ref, bk_ref, wv_ref, bv_ref, ws_ref, bs_ref,
               q0_ref, q1_ref, k0_ref, k1_ref, v0_ref, v1_ref,
               ef_ref, skip_ref):
    dt = dt_ref[0, 0]
    x = x_ref[...]
    xdt = xdt_ref[...]

    px, py, pz = x[:, 1:2], x[:, 2:3], x[:, 3:4]
    dx = xdt[:, 1:2] - px
    dy = xdt[:, 2:3] - py
    dz = xdt[:, 3:4] - pz
    nrm = jnp.sqrt(dx * dx + dy * dy + dz * dz)
    inv_n = 1.0 / jnp.maximum(nrm, 1e-12)
    vx = dx * inv_n / dt
    vy = dy * inv_n / dt
    vz = dz * inv_n / dt

    rx = mp_ref[0, 0] - px
    ry = mp_ref[0, 1] - py
    rz = mp_ref[0, 2] - pz
    rn = jnp.sqrt(rx * rx + ry * ry + rz * rz)
    dist_score = 1.0 / (rn + 1e-6)
    n1 = jnp.maximum(rn, 1e-6)
    vn = jnp.sqrt(vx * vx + vy * vy + vz * vz)
    n2 = jnp.maximum(vn, 1e-6)
    dir_score = (rx * vx + ry * vy + rz * vz) / (n1 * n2)

    we = we_ref[...]
    ef = (dist_score * we[0:1, :] + dir_score * we[1:2, :] + nrm * we[2:3, :]
          + be_ref[...])
    ef_ref[...] = ef

    xe = jnp.dot(x, wn_ref[...], preferred_element_type=jnp.float32) + bn_ref[...]
    q = jnp.dot(xe, wq_ref[...], preferred_element_type=jnp.float32) + bq_ref[...]
    k = jnp.dot(xe, wk_ref[...], preferred_element_type=jnp.float32) + bk_ref[...]
    v = jnp.dot(xe, wv_ref[...], preferred_element_type=jnp.float32) + bv_ref[...]
    skip_ref[...] = (jnp.dot(xe, ws_ref[...], preferred_element_type=jnp.float32)
                     + bs_ref[...])
    q0_ref[...] = q[:, :HALF]
    q1_ref[...] = q[:, HALF:]
    k0_ref[...] = k[:, :HALF]
    k1_ref[...] = k[:, HALF:]
    v0_ref[...] = v[:, :HALF]
    v1_ref[...] = v[:, HALF:]


def _node_precompute(dt_arr, mp, x8, xdt8, wn8, bn, we, be, wq, bq, wk, bk,
                     wv, bv, ws, bs):
    f32 = jnp.float32
    row = lambda i: (i, 0)
    fix = lambda i: (0, 0)
    smem = pl.BlockSpec(memory_space=pltpu.SMEM)
    out16 = pl.BlockSpec((BLK, HALF), row)
    out32 = pl.BlockSpec((BLK, D), row)
    return pl.pallas_call(
        _node_body,
        grid=(N // BLK,),
        in_specs=[
            smem, smem,
            pl.BlockSpec((BLK, 8), row), pl.BlockSpec((BLK, 8), row),
            pl.BlockSpec((8, D), fix), pl.BlockSpec((1, D), fix),
            pl.BlockSpec((3, D), fix), pl.BlockSpec((1, D), fix),
            pl.BlockSpec((D, D), fix), pl.BlockSpec((1, D), fix),
            pl.BlockSpec((D, D), fix), pl.BlockSpec((1, D), fix),
            pl.BlockSpec((D, D), fix), pl.BlockSpec((1, D), fix),
            pl.BlockSpec((D, D), fix), pl.BlockSpec((1, D), fix),
        ],
        out_specs=[out16] * 6 + [out32, out32],
        out_shape=[jax.ShapeDtypeStruct((N, HALF), f32)] * 6
        + [jax.ShapeDtypeStruct((N, D), f32)] * 2,
    )(dt_arr, mp, x8, xdt8, wn8, bn, we, be, wq, bq, wk, bk, wv, bv, ws, bs)


def _edge_body(q0, q1, k0, k1, v0, v1, src_h, tgt_h, zm, zd,
               msg_out, den_out,
               src_v, tgt_v, q_v, k_v, v_v, msg_v, ex0_v, ex1_v,
               msg_acc, den_acc0, den_acc1):
    cid = lax.axis_index("c")
    sid = lax.axis_index("s")

    r0 = sid * ROWS_T
    pltpu.sync_copy(zm.at[pl.ds(r0, ROWS_T)], msg_acc.at[pl.ds(r0, ROWS_T)])
    pltpu.sync_copy(zd.at[pl.ds(r0, ROWS_T)], den_acc0.at[pl.ds(r0, ROWS_T)])
    pltpu.sync_copy(zd.at[pl.ds(r0, ROWS_T)], den_acc1.at[pl.ds(r0, ROWS_T)])
    plsc.subcore_barrier()

    iota = lax.iota(jnp.int32, LANES)

    def chunk_body(g, carry):
        base = g * ECHUNK
        pltpu.sync_copy(src_h.at[pl.ds(base, ECHUNK)], src_v)
        pltpu.sync_copy(tgt_h.at[pl.ds(base, ECHUNK)], tgt_v)

        @pl.when(cid == 0)
        def _():
            pltpu.sync_copy(q0.at[tgt_v], q_v)
            pltpu.sync_copy(k0.at[src_v], k_v)
            pltpu.sync_copy(v0.at[src_v], v_v)

        @pl.when(cid == 1)
        def _():
            pltpu.sync_copy(q1.at[tgt_v], q_v)
            pltpu.sync_copy(k1.at[src_v], k_v)
            pltpu.sync_copy(v1.at[src_v], v_v)

        def grp(i, c2):
            rows = i * LANES + iota
            acc0 = jnp.zeros((LANES,), jnp.float32)
            acc1 = jnp.zeros((LANES,), jnp.float32)
            for j in range(HALF):
                col = jnp.full((LANES,), j, jnp.int32)
                qv = plsc.load_gather(q_v, [rows, col])
                kv = plsc.load_gather(k_v, [rows, col])
                if j < C:
                    acc0 = acc0 + qv * kv
                else:
                    acc1 = acc1 + qv * kv
            ex0 = jnp.exp(acc0 * SCALE)
            ex1 = jnp.exp(acc1 * SCALE)
            ex0_v[pl.ds(i * LANES, LANES)] = ex0
            ex1_v[pl.ds(i * LANES, LANES)] = ex1
            for ch in range(HALF):
                col = jnp.full((LANES,), ch, jnp.int32)
                vv = plsc.load_gather(v_v, [rows, col])
                m = vv * (ex0 if ch < C else ex1)
                plsc.store_scatter(msg_v, [rows, col], m)
            return c2

        lax.fori_loop(0, ECHUNK // LANES, grp, 0)

        pltpu.sync_copy(msg_v, msg_acc.at[tgt_v], add=True)
        pltpu.sync_copy(ex0_v, den_acc0.at[tgt_v], add=True)
        pltpu.sync_copy(ex1_v, den_acc1.at[tgt_v], add=True)
        return carry

    c_lo = sid * NCHUNKS // NSUB
    c_hi = (sid + 1) * NCHUNKS // NSUB
    lax.fori_loop(c_lo, c_hi, chunk_body, 0)

    plsc.subcore_barrier()

    @pl.when(cid == 0)
    def _():
        pltpu.sync_copy(msg_acc.at[pl.ds(r0, ROWS_T)],
                        msg_out.at[0, pl.ds(r0, ROWS_T)])
        pltpu.sync_copy(den_acc0.at[pl.ds(r0, ROWS_T)],
                        den_out.at[0, 0, pl.ds(r0, ROWS_T)])
        pltpu.sync_copy(den_acc1.at[pl.ds(r0, ROWS_T)],
                        den_out.at[0, 1, pl.ds(r0, ROWS_T)])

    @pl.when(cid == 1)
    def _():
        pltpu.sync_copy(msg_acc.at[pl.ds(r0, ROWS_T)],
                        msg_out.at[1, pl.ds(r0, ROWS_T)])
        pltpu.sync_copy(den_acc0.at[pl.ds(r0, ROWS_T)],
                        den_out.at[1, 0, pl.ds(r0, ROWS_T)])
        pltpu.sync_copy(den_acc1.at[pl.ds(r0, ROWS_T)],
                        den_out.at[1, 1, pl.ds(r0, ROWS_T)])


def _edge_pass(q0, q1, k0, k1, v0, v1, src, tgt, zm, zd):
    f32 = jnp.float32
    kern = functools.partial(
        pl.kernel,
        out_type=(jax.ShapeDtypeStruct((NSC, N_PAD, HALF), f32),
                  jax.ShapeDtypeStruct((NSC, 2, N_PAD), f32)),
        mesh=plsc.VectorSubcoreMesh(core_axis_name="c", subcore_axis_name="s"),
        compiler_params=pltpu.CompilerParams(needs_layout_passes=False,
                                             use_tc_tiling_on_sc=False),
        scratch_types=[
            pltpu.VMEM((ECHUNK,), jnp.int32),
            pltpu.VMEM((ECHUNK,), jnp.int32),
            pltpu.VMEM((ECHUNK, HALF), f32),
            pltpu.VMEM((ECHUNK, HALF), f32),
            pltpu.VMEM((ECHUNK, HALF), f32),
            pltpu.VMEM((ECHUNK, HALF), f32),
            pltpu.VMEM((ECHUNK,), f32),
            pltpu.VMEM((ECHUNK,), f32),
            pltpu.VMEM_SHARED((N_PAD, HALF), f32),
            pltpu.VMEM_SHARED((N_PAD,), f32),
            pltpu.VMEM_SHARED((N_PAD,), f32),
        ],
    )(_edge_body)
    return kern(q0, q1, k0, k1, v0, v1, src, tgt, zm, zd)


def _combine_body(bc_ref, msg_ref, den_ref, ef_ref, skip_ref, wc_ref, s_ref):
    msg = jnp.concatenate([msg_ref[0], msg_ref[1]], axis=1)
    den = den_ref[...]
    invd = 1.0 / (den + 1e-16)
    sattn = den * invd
    ef = ef_ref[...]
    parts = []
    for h in range(H):
        parts.append(msg[:, h * C:(h + 1) * C] * invd[:, h:h + 1]
                     + ef[:, h * C:(h + 1) * C] * sattn[:, h:h + 1])
    x_trans = jnp.concatenate(parts, axis=1) + skip_ref[...]
    s_ref[...] = (jnp.dot(x_trans, wc_ref[...],
                          preferred_element_type=jnp.float32)
                  + 0.5 * bc_ref[0, 0])


def _combine(bc, msg, den, ef, skip, wc):
    return pl.pallas_call(
        _combine_body,
        grid=(N // BLK,),
        in_specs=[
            pl.BlockSpec(memory_space=pltpu.SMEM),
            pl.BlockSpec((NSC, BLK, HALF), lambda i: (0, i, 0)),
            pl.BlockSpec((BLK, 4), lambda i: (i, 0)),
            pl.BlockSpec((BLK, D), lambda i: (i, 0)),
            pl.BlockSpec((BLK, D), lambda i: (i, 0)),
            pl.BlockSpec((D, 1), lambda i: (0, 0)),
        ],
        out_specs=pl.BlockSpec((BLK, 1), lambda i: (i, 0)),
        out_shape=jax.ShapeDtypeStruct((N, 1), jnp.float32),
    )(bc, msg, den, ef, skip, wc)


def _edge_out_body(s_h, src_h, tgt_h, out_h, s_v, src_v, tgt_v, out_v):
    cid = lax.axis_index("c")
    sid = lax.axis_index("s")
    wid = sid * NSC + cid
    pltpu.sync_copy(s_h, s_v)

    def chunk_body(g, carry):
        base = wid * EPW + g * OCHUNK
        pltpu.sync_copy(src_h.at[pl.ds(base, OCHUNK)], src_v)
        pltpu.sync_copy(tgt_h.at[pl.ds(base, OCHUNK)], tgt_v)

        def grp(i, c2):
            o = i * LANES
            si = src_v[pl.ds(o, LANES)]
            ti = tgt_v[pl.ds(o, LANES)]
            sv = plsc.load_gather(s_v, [si])
            tv = plsc.load_gather(s_v, [ti])
            x = sv + tv
            out_v[pl.ds(o, LANES)] = 1.0 / (1.0 + jnp.exp(-x))
            return c2

        lax.fori_loop(0, OCHUNK // LANES, grp, 0)
        pltpu.sync_copy(out_v, out_h.at[pl.ds(base, OCHUNK)])
        return carry

    lax.fori_loop(0, EPW // OCHUNK, chunk_body, 0)


def _edge_out(s, src, tgt):
    return pl.kernel(
        _edge_out_body,
        out_type=jax.ShapeDtypeStruct((E,), jnp.float32),
        mesh=plsc.VectorSubcoreMesh(core_axis_name="c", subcore_axis_name="s"),
        compiler_params=pltpu.CompilerParams(needs_layout_passes=False),
        scratch_types=[
            pltpu.VMEM((N,), jnp.float32),
            pltpu.VMEM((OCHUNK,), jnp.int32),
            pltpu.VMEM((OCHUNK,), jnp.int32),
            pltpu.VMEM((OCHUNK,), jnp.float32),
        ],
    )(s, src, tgt)


def kernel(x_t, x_t_dt, edge_index, dt, W_node, b_node, W_edge, b_edge,
           W_q, b_q, W_k, b_k, W_v, b_v, W_skip, b_skip, W_cls, b_cls):
    f32 = jnp.float32
    dt_arr = jnp.reshape(jnp.asarray(dt, f32), (1, 1))
    mp = jnp.pad(x_t[0:1, 1:4], ((0, 0), (0, 1)))
    x8 = jnp.pad(x_t, ((0, 0), (0, 1)))
    xdt8 = jnp.pad(x_t_dt, ((0, 0), (0, 1)))
    wn8 = jnp.pad(W_node, ((0, 1), (0, 0)))
    r = lambda b: jnp.reshape(b, (1, D))

    q0, q1, k0, k1, v0, v1, ef, skip = _node_precompute(
        dt_arr, mp, x8, xdt8, wn8, r(b_node), W_edge, r(b_edge),
        W_q, r(b_q), W_k, r(b_k), W_v, r(b_v), W_skip, r(b_skip))

    src = edge_index[0]
    tgt = edge_index[1]
    zm = jnp.zeros((N_PAD, HALF), f32)
    zd = jnp.zeros((N_PAD,), f32)
    msg, den = _edge_pass(q0, q1, k0, k1, v0, v1, src, tgt, zm, zd)
    msg = msg[:, :N, :]
    den = jnp.transpose(jnp.reshape(den, (H, N_PAD))[:, :N])

    bc = jnp.reshape(b_cls, (1, 1))
    s = _combine(bc, msg, den, ef, skip, W_cls)

    return _edge_out(jnp.reshape(s, (N,)), src, tgt)


# speedup vs baseline: 73.7460x; 1.5506x over previous
import functools

import jax
import jax.numpy as jnp
from jax import lax
from jax.experimental import pallas as pl
from jax.experimental.pallas import tpu as pltpu
from jax.experimental.pallas import tpu_sc as plsc

N = 100000
E = 1600000
H = 4
D = 32
C = D // H
HALF = 16
SCALE = 1.0 / (C ** 0.5)

BLK = 4000

NSC = 2
NSUB = 16
LANES = 16

ECHUNK = 160
EPT = E // NSUB
NCH_T = EPT // ECHUNK
ROWS_T = 6256
N_PAD = ROWS_T * NSUB

OCHUNK = 2000
EPW = E // (NSC * NSUB)


def _node_body(dt_ref, mp_ref, x_ref, xdt_ref, wn_ref, bn_ref, we_ref, be_ref,
               wq_ref, bq_ref, wk_ref, bk_ref, wv_ref, bv_ref, ws_ref, bs_ref,
               q0_ref, q1_ref, k0_ref, k1_ref, v0_ref, v1_ref,
               ef_ref, skip_ref):
    dt = dt_ref[0, 0]
    x = x_ref[...]
    xdt = xdt_ref[...]

    px, py, pz = x[:, 1:2], x[:, 2:3], x[:, 3:4]
    dx = xdt[:, 1:2] - px
    dy = xdt[:, 2:3] - py
    dz = xdt[:, 3:4] - pz
    nrm = jnp.sqrt(dx * dx + dy * dy + dz * dz)
    inv_n = 1.0 / jnp.maximum(nrm, 1e-12)
    vx = dx * inv_n / dt
    vy = dy * inv_n / dt
    vz = dz * inv_n / dt

    rx = mp_ref[0, 0] - px
    ry = mp_ref[0, 1] - py
    rz = mp_ref[0, 2] - pz
    rn = jnp.sqrt(rx * rx + ry * ry + rz * rz)
    dist_score = 1.0 / (rn + 1e-6)
    n1 = jnp.maximum(rn, 1e-6)
    vn = jnp.sqrt(vx * vx + vy * vy + vz * vz)
    n2 = jnp.maximum(vn, 1e-6)
    dir_score = (rx * vx + ry * vy + rz * vz) / (n1 * n2)

    we = we_ref[...]
    ef = (dist_score * we[0:1, :] + dir_score * we[1:2, :] + nrm * we[2:3, :]
          + be_ref[...])
    ef_ref[...] = ef

    xe = jnp.dot(x, wn_ref[...], preferred_element_type=jnp.float32) + bn_ref[...]
    q = jnp.dot(xe, wq_ref[...], preferred_element_type=jnp.float32) + bq_ref[...]
    k = jnp.dot(xe, wk_ref[...], preferred_element_type=jnp.float32) + bk_ref[...]
    v = jnp.dot(xe, wv_ref[...], preferred_element_type=jnp.float32) + bv_ref[...]
    skip_ref[...] = (jnp.dot(xe, ws_ref[...], preferred_element_type=jnp.float32)
                     + bs_ref[...])
    q0_ref[...] = q[:, :HALF]
    q1_ref[...] = q[:, HALF:]
    k0_ref[...] = k[:, :HALF]
    k1_ref[...] = k[:, HALF:]
    v0_ref[...] = v[:, :HALF]
    v1_ref[...] = v[:, HALF:]


def _node_precompute(dt_arr, mp, x8, xdt8, wn8, bn, we, be, wq, bq, wk, bk,
                     wv, bv, ws, bs):
    f32 = jnp.float32
    row = lambda i: (i, 0)
    fix = lambda i: (0, 0)
    smem = pl.BlockSpec(memory_space=pltpu.SMEM)
    out16 = pl.BlockSpec((BLK, HALF), row)
    out32 = pl.BlockSpec((BLK, D), row)
    return pl.pallas_call(
        _node_body,
        grid=(N // BLK,),
        in_specs=[
            smem, smem,
            pl.BlockSpec((BLK, 8), row), pl.BlockSpec((BLK, 8), row),
            pl.BlockSpec((8, D), fix), pl.BlockSpec((1, D), fix),
            pl.BlockSpec((3, D), fix), pl.BlockSpec((1, D), fix),
            pl.BlockSpec((D, D), fix), pl.BlockSpec((1, D), fix),
            pl.BlockSpec((D, D), fix), pl.BlockSpec((1, D), fix),
            pl.BlockSpec((D, D), fix), pl.BlockSpec((1, D), fix),
            pl.BlockSpec((D, D), fix), pl.BlockSpec((1, D), fix),
        ],
        out_specs=[out16] * 6 + [out32, out32],
        out_shape=[jax.ShapeDtypeStruct((N, HALF), f32)] * 6
        + [jax.ShapeDtypeStruct((N, D), f32)] * 2,
    )(dt_arr, mp, x8, xdt8, wn8, bn, we, be, wq, bq, wk, bk, wv, bv, ws, bs)


def _edge_body(q0, q1, k0, k1, v0, v1, src_h, tgt_h, zm, zd,
               msg_out, den_out,
               src_a, src_b, tgt_a, tgt_b, q_v, k_v, v_v,
               msg_a, msg_b, ex0_a, ex0_b, ex1_a, ex1_b,
               msg_acc, den_acc0, den_acc1, sem_i, sem_g, sem_s):
    cid = lax.axis_index("c")
    sid = lax.axis_index("s")

    r0 = sid * ROWS_T
    pltpu.sync_copy(zm.at[pl.ds(r0, ROWS_T)], msg_acc.at[pl.ds(r0, ROWS_T)])
    pltpu.sync_copy(zd.at[pl.ds(r0, ROWS_T)], den_acc0.at[pl.ds(r0, ROWS_T)])
    pltpu.sync_copy(zd.at[pl.ds(r0, ROWS_T)], den_acc1.at[pl.ds(r0, ROWS_T)])
    plsc.subcore_barrier()

    iota = lax.iota(jnp.int32, LANES)
    e0 = sid * EPT
    srcs = (src_a, src_b)
    tgts = (tgt_a, tgt_b)
    msgs = (msg_a, msg_b)
    ex0s = (ex0_a, ex0_b)
    ex1s = (ex1_a, ex1_b)

    def start_idx(g, pb):
        gg = jnp.where(g < NCH_T, g, 0)
        base = e0 + gg * ECHUNK
        pltpu.async_copy(src_h.at[pl.ds(base, ECHUNK)], srcs[pb], sem_i)
        pltpu.async_copy(tgt_h.at[pl.ds(base, ECHUNK)], tgts[pb], sem_i)

    def wait_idx(pb):
        pltpu.make_async_copy(src_h.at[pl.ds(0, ECHUNK)], srcs[pb], sem_i).wait()
        pltpu.make_async_copy(tgt_h.at[pl.ds(0, ECHUNK)], tgts[pb], sem_i).wait()

    def start_gathers(pb):
        @pl.when(cid == 0)
        def _():
            pltpu.async_copy(q0.at[tgts[pb]], q_v, sem_g)
            pltpu.async_copy(k0.at[srcs[pb]], k_v, sem_g)
            pltpu.async_copy(v0.at[srcs[pb]], v_v, sem_g)

        @pl.when(cid == 1)
        def _():
            pltpu.async_copy(q1.at[tgts[pb]], q_v, sem_g)
            pltpu.async_copy(k1.at[srcs[pb]], k_v, sem_g)
            pltpu.async_copy(v1.at[srcs[pb]], v_v, sem_g)

    def wait_gathers(pb):
        pltpu.make_async_copy(q0.at[tgts[pb]], q_v, sem_g).wait()
        pltpu.make_async_copy(k0.at[srcs[pb]], k_v, sem_g).wait()
        pltpu.make_async_copy(v0.at[srcs[pb]], v_v, sem_g).wait()

    def start_scatters(pb):
        pltpu.async_copy(msgs[pb], msg_acc.at[tgts[pb]], sem_s, add=True)
        pltpu.async_copy(ex0s[pb], den_acc0.at[tgts[pb]], sem_s, add=True)
        pltpu.async_copy(ex1s[pb], den_acc1.at[tgts[pb]], sem_s, add=True)

    def wait_scatters(pb):
        pltpu.make_async_copy(msgs[pb], msg_acc.at[tgts[pb]], sem_s).wait()
        pltpu.make_async_copy(ex0s[pb], den_acc0.at[tgts[pb]], sem_s).wait()
        pltpu.make_async_copy(ex1s[pb], den_acc1.at[tgts[pb]], sem_s).wait()

    def compute(pb):
        msg_v = msgs[pb]
        ex0_v = ex0s[pb]
        ex1_v = ex1s[pb]

        def grp(i, c2):
            rows = i * LANES + iota
            acc0 = jnp.zeros((LANES,), jnp.float32)
            acc1 = jnp.zeros((LANES,), jnp.float32)
            for j in range(HALF):
                col = jnp.full((LANES,), j, jnp.int32)
                qv = plsc.load_gather(q_v, [rows, col])
                kv = plsc.load_gather(k_v, [rows, col])
                if j < C:
                    acc0 = acc0 + qv * kv
                else:
                    acc1 = acc1 + qv * kv
            ex0 = jnp.exp(acc0 * SCALE)
            ex1 = jnp.exp(acc1 * SCALE)
            ex0_v[pl.ds(i * LANES, LANES)] = ex0
            ex1_v[pl.ds(i * LANES, LANES)] = ex1
            for ch in range(HALF):
                col = jnp.full((LANES,), ch, jnp.int32)
                vv = plsc.load_gather(v_v, [rows, col])
                m = vv * (ex0 if ch < C else ex1)
                plsc.store_scatter(msg_v, [rows, col], m)
            return c2

        lax.fori_loop(0, ECHUNK // LANES, grp, 0)

    start_idx(jnp.int32(0), 0)

    def pair_body(t, carry):
        for b in range(2):
            g = t * 2 + b

            @pl.when(g > 0)
            def _():
                wait_scatters(1 - b)
            start_idx(g + 1, 1 - b)
            wait_idx(b)
            start_gathers(b)
            wait_gathers(b)
            compute(b)
            start_scatters(b)
        return carry

    lax.fori_loop(0, NCH_T // 2, pair_body, 0)
    if NCH_T % 2:
        wait_scatters(1)
        wait_idx(0)
        start_gathers(0)
        wait_gathers(0)
        compute(0)
        start_scatters(0)
        wait_scatters(0)
    else:
        wait_idx(0)
        wait_scatters(1)

    plsc.subcore_barrier()

    @pl.when(cid == 0)
    def _():
        pltpu.sync_copy(msg_acc.at[pl.ds(r0, ROWS_T)],
                        msg_out.at[0, pl.ds(r0, ROWS_T)])
        pltpu.sync_copy(den_acc0.at[pl.ds(r0, ROWS_T)],
                        den_out.at[0, 0, pl.ds(r0, ROWS_T)])
        pltpu.sync_copy(den_acc1.at[pl.ds(r0, ROWS_T)],
                        den_out.at[0, 1, pl.ds(r0, ROWS_T)])

    @pl.when(cid == 1)
    def _():
        pltpu.sync_copy(msg_acc.at[pl.ds(r0, ROWS_T)],
                        msg_out.at[1, pl.ds(r0, ROWS_T)])
        pltpu.sync_copy(den_acc0.at[pl.ds(r0, ROWS_T)],
                        den_out.at[1, 0, pl.ds(r0, ROWS_T)])
        pltpu.sync_copy(den_acc1.at[pl.ds(r0, ROWS_T)],
                        den_out.at[1, 1, pl.ds(r0, ROWS_T)])


def _edge_pass(q0, q1, k0, k1, v0, v1, src, tgt, zm, zd):
    f32 = jnp.float32
    kern = functools.partial(
        pl.kernel,
        out_type=(jax.ShapeDtypeStruct((NSC, N_PAD, HALF), f32),
                  jax.ShapeDtypeStruct((NSC, 2, N_PAD), f32)),
        mesh=plsc.VectorSubcoreMesh(core_axis_name="c", subcore_axis_name="s"),
        compiler_params=pltpu.CompilerParams(needs_layout_passes=False,
                                             use_tc_tiling_on_sc=False),
        scratch_types=[
            pltpu.VMEM((ECHUNK,), jnp.int32),
            pltpu.VMEM((ECHUNK,), jnp.int32),
            pltpu.VMEM((ECHUNK,), jnp.int32),
            pltpu.VMEM((ECHUNK,), jnp.int32),
            pltpu.VMEM((ECHUNK, HALF), f32),
            pltpu.VMEM((ECHUNK, HALF), f32),
            pltpu.VMEM((ECHUNK, HALF), f32),
            pltpu.VMEM((ECHUNK, HALF), f32),
            pltpu.VMEM((ECHUNK, HALF), f32),
            pltpu.VMEM((ECHUNK,), f32),
            pltpu.VMEM((ECHUNK,), f32),
            pltpu.VMEM((ECHUNK,), f32),
            pltpu.VMEM((ECHUNK,), f32),
            pltpu.VMEM_SHARED((N_PAD, HALF), f32),
            pltpu.VMEM_SHARED((N_PAD,), f32),
            pltpu.VMEM_SHARED((N_PAD,), f32),
            pltpu.SemaphoreType.DMA,
            pltpu.SemaphoreType.DMA,
            pltpu.SemaphoreType.DMA,
        ],
    )(_edge_body)
    return kern(q0, q1, k0, k1, v0, v1, src, tgt, zm, zd)


def _combine_body(bc_ref, msg_ref, den_ref, ef_ref, skip_ref, wc_ref, s_ref):
    msg = jnp.concatenate([msg_ref[0], msg_ref[1]], axis=1)
    den = den_ref[...]
    invd = 1.0 / (den + 1e-16)
    sattn = den * invd
    ef = ef_ref[...]
    parts = []
    for h in range(H):
        parts.append(msg[:, h * C:(h + 1) * C] * invd[:, h:h + 1]
                     + ef[:, h * C:(h + 1) * C] * sattn[:, h:h + 1])
    x_trans = jnp.concatenate(parts, axis=1) + skip_ref[...]
    s_ref[...] = (jnp.dot(x_trans, wc_ref[...],
                          preferred_element_type=jnp.float32)
                  + 0.5 * bc_ref[0, 0])


def _combine(bc, msg, den, ef, skip, wc):
    return pl.pallas_call(
        _combine_body,
        grid=(N // BLK,),
        in_specs=[
            pl.BlockSpec(memory_space=pltpu.SMEM),
            pl.BlockSpec((NSC, BLK, HALF), lambda i: (0, i, 0)),
            pl.BlockSpec((BLK, 4), lambda i: (i, 0)),
            pl.BlockSpec((BLK, D), lambda i: (i, 0)),
            pl.BlockSpec((BLK, D), lambda i: (i, 0)),
            pl.BlockSpec((D, 1), lambda i: (0, 0)),
        ],
        out_specs=pl.BlockSpec((BLK, 1), lambda i: (i, 0)),
        out_shape=jax.ShapeDtypeStruct((N, 1), jnp.float32),
    )(bc, msg, den, ef, skip, wc)


def _edge_out_body(s_h, src_h, tgt_h, out_h, s_v, src_v, tgt_v, out_v):
    cid = lax.axis_index("c")
    sid = lax.axis_index("s")
    wid = sid * NSC + cid
    pltpu.sync_copy(s_h, s_v)

    def chunk_body(g, carry):
        base = wid * EPW + g * OCHUNK
        pltpu.sync_copy(src_h.at[pl.ds(base, OCHUNK)], src_v)
        pltpu.sync_copy(tgt_h.at[pl.ds(base, OCHUNK)], tgt_v)

        def grp(i, c2):
            o = i * LANES
            si = src_v[pl.ds(o, LANES)]
            ti = tgt_v[pl.ds(o, LANES)]
            sv = plsc.load_gather(s_v, [si])
            tv = plsc.load_gather(s_v, [ti])
            x = sv + tv
            out_v[pl.ds(o, LANES)] = 1.0 / (1.0 + jnp.exp(-x))
            return c2

        lax.fori_loop(0, OCHUNK // LANES, grp, 0)
        pltpu.sync_copy(out_v, out_h.at[pl.ds(base, OCHUNK)])
        return carry

    lax.fori_loop(0, EPW // OCHUNK, chunk_body, 0)


def _edge_out(s, src, tgt):
    return pl.kernel(
        _edge_out_body,
        out_type=jax.ShapeDtypeStruct((E,), jnp.float32),
        mesh=plsc.VectorSubcoreMesh(core_axis_name="c", subcore_axis_name="s"),
        compiler_params=pltpu.CompilerParams(needs_layout_passes=False),
        scratch_types=[
            pltpu.VMEM((N,), jnp.float32),
            pltpu.VMEM((OCHUNK,), jnp.int32),
            pltpu.VMEM((OCHUNK,), jnp.int32),
            pltpu.VMEM((OCHUNK,), jnp.float32),
        ],
    )(s, src, tgt)


def kernel(x_t, x_t_dt, edge_index, dt, W_node, b_node, W_edge, b_edge,
           W_q, b_q, W_k, b_k, W_v, b_v, W_skip, b_skip, W_cls, b_cls):
    f32 = jnp.float32
    dt_arr = jnp.reshape(jnp.asarray(dt, f32), (1, 1))
    mp = jnp.pad(x_t[0:1, 1:4], ((0, 0), (0, 1)))
    x8 = jnp.pad(x_t, ((0, 0), (0, 1)))
    xdt8 = jnp.pad(x_t_dt, ((0, 0), (0, 1)))
    wn8 = jnp.pad(W_node, ((0, 1), (0, 0)))
    r = lambda b: jnp.reshape(b, (1, D))

    q0, q1, k0, k1, v0, v1, ef, skip = _node_precompute(
        dt_arr, mp, x8, xdt8, wn8, r(b_node), W_edge, r(b_edge),
        W_q, r(b_q), W_k, r(b_k), W_v, r(b_v), W_skip, r(b_skip))

    src = edge_index[0]
    tgt = edge_index[1]
    zm = jnp.zeros((N_PAD, HALF), f32)
    zd = jnp.zeros((N_PAD,), f32)
    msg, den = _edge_pass(q0, q1, k0, k1, v0, v1, src, tgt, zm, zd)
    msg = msg[:, :N, :]
    den = jnp.transpose(jnp.reshape(den, (H, N_PAD))[:, :N])

    bc = jnp.reshape(b_cls, (1, 1))
    s = _combine(bc, msg, den, ef, skip, W_cls)

    return _edge_out(jnp.reshape(s, (N,)), src, tgt)


# speedup vs baseline: 78.7123x; 1.0673x over previous
import functools

import jax
import jax.numpy as jnp
from jax import lax
from jax.experimental import pallas as pl
from jax.experimental.pallas import tpu as pltpu
from jax.experimental.pallas import tpu_sc as plsc

N = 100000
E = 1600000
H = 4
D = 32
C = D // H
HALF = 16
SCALE = 1.0 / (C ** 0.5)

BLK = 4000
BLKA = 4352

NSC = 2
NSUB = 16
LANES = 16

ECHUNK = 128
NCH_T = 786
EPT = NCH_T * ECHUNK
E_PAD = EPT * NSUB
ROWS_T = 6256
N_PAD = ROWS_T * NSUB

OCHUNK = 2000
EPW = E // (NSC * NSUB)


def _node_body(dt_ref, mp_ref, xt_ref, xdtt_ref, wn_ref, bn_ref, we_ref,
               be_ref, wq_ref, bq_ref, wk_ref, bk_ref, wv_ref, bv_ref,
               ws_ref, bs_ref,
               q0_ref, q1_ref, kv0_ref, kv1_ref, ef_ref, skip_ref):
    dt = dt_ref[0, 0]
    xt = xt_ref[...]
    xdtt = xdtt_ref[...]

    pos = xt[1:4, :]
    d3 = xdtt[1:4, :] - pos
    nrm = jnp.sqrt(jnp.sum(d3 * d3, axis=0, keepdims=True))
    vel = d3 / jnp.maximum(nrm, 1e-12) / dt
    mp = mp_ref[...]
    rel = mp - pos
    rn = jnp.sqrt(jnp.sum(rel * rel, axis=0, keepdims=True))
    dist_score = 1.0 / (rn + 1e-6)
    n1 = jnp.maximum(rn, 1e-6)
    vn = jnp.sqrt(jnp.sum(vel * vel, axis=0, keepdims=True))
    n2 = jnp.maximum(vn, 1e-6)
    dir_score = jnp.sum(rel * vel, axis=0, keepdims=True) / (n1 * n2)
    ea = jnp.concatenate([dist_score, dir_score, nrm], axis=0)

    f32 = jnp.float32
    dn = (((0,), (0,)), ((), ()))
    ef = lax.dot_general(ea, we_ref[...], dn, preferred_element_type=f32)
    ef_ref[...] = ef + be_ref[...]
    xe = (lax.dot_general(xt, wn_ref[...], dn, preferred_element_type=f32)
          + bn_ref[...])
    q = jnp.dot(xe, wq_ref[...], preferred_element_type=f32) + bq_ref[...]
    k = jnp.dot(xe, wk_ref[...], preferred_element_type=f32) + bk_ref[...]
    v = jnp.dot(xe, wv_ref[...], preferred_element_type=f32) + bv_ref[...]
    skip_ref[...] = (jnp.dot(xe, ws_ref[...], preferred_element_type=f32)
                     + bs_ref[...])
    q0_ref[...] = q[:, :HALF]
    q1_ref[...] = q[:, HALF:]
    kv0_ref[...] = jnp.concatenate([k[:, :HALF], v[:, :HALF]], axis=1)
    kv1_ref[...] = jnp.concatenate([k[:, HALF:], v[:, HALF:]], axis=1)


def _node_precompute(dt_arr, mp, xT, xdtT, wn, bn, we, be, wq, bq, wk, bk,
                     wv, bv, ws, bs):
    f32 = jnp.float32
    row = lambda i: (i, 0)
    col = lambda i: (0, i)
    fix = lambda i: (0, 0)
    smem = pl.BlockSpec(memory_space=pltpu.SMEM)
    out16 = pl.BlockSpec((BLKA, HALF), row)
    out32 = pl.BlockSpec((BLKA, D), row)
    return pl.pallas_call(
        _node_body,
        grid=(N_PAD // BLKA,),
        in_specs=[
            smem, pl.BlockSpec((3, 1), fix),
            pl.BlockSpec((7, BLKA), col), pl.BlockSpec((7, BLKA), col),
            pl.BlockSpec((7, D), fix), pl.BlockSpec((1, D), fix),
            pl.BlockSpec((3, D), fix), pl.BlockSpec((1, D), fix),
            pl.BlockSpec((D, D), fix), pl.BlockSpec((1, D), fix),
            pl.BlockSpec((D, D), fix), pl.BlockSpec((1, D), fix),
            pl.BlockSpec((D, D), fix), pl.BlockSpec((1, D), fix),
            pl.BlockSpec((D, D), fix), pl.BlockSpec((1, D), fix),
        ],
        out_specs=[out16, out16, out32, out32, out32, out32],
        out_shape=[jax.ShapeDtypeStruct((N_PAD, HALF), f32)] * 2
        + [jax.ShapeDtypeStruct((N_PAD, D), f32)] * 4,
    )(dt_arr, mp, xT, xdtT, wn, bn, we, be, wq, bq, wk, bk, wv, bv, ws, bs)


def _edge_body(q0, q1, kv0, kv1, src_h, tgt_h, zm, zd,
               msg_out, den_out,
               src_a, src_b, src_c, tgt_a, tgt_b, tgt_c,
               qv_a, qv_b, kv_a, kv_b,
               msg_a, msg_b, ex0_a, ex0_b, ex1_a, ex1_b,
               msg_acc, den_acc0, den_acc1, sem_i, sem_g, sem_s):
    cid = lax.axis_index("c")
    sid = lax.axis_index("s")

    r0 = sid * ROWS_T
    pltpu.sync_copy(zm, msg_acc.at[pl.ds(r0, ROWS_T)])
    pltpu.sync_copy(zd, den_acc0.at[pl.ds(r0, ROWS_T)])
    pltpu.sync_copy(zd, den_acc1.at[pl.ds(r0, ROWS_T)])
    plsc.subcore_barrier()

    iota = lax.iota(jnp.int32, LANES)
    e0 = sid * EPT
    srcs = (src_a, src_b, src_c)
    tgts = (tgt_a, tgt_b, tgt_c)
    qvs = (qv_a, qv_b)
    kvs = (kv_a, kv_b)
    msgs = (msg_a, msg_b)
    ex0s = (ex0_a, ex0_b)
    ex1s = (ex1_a, ex1_b)

    def start_idx(g, j):
        gg = jnp.where(g < NCH_T, g, 0)
        base = e0 + gg * ECHUNK
        pltpu.async_copy(src_h.at[pl.ds(base, ECHUNK)], srcs[j], sem_i)
        pltpu.async_copy(tgt_h.at[pl.ds(base, ECHUNK)], tgts[j], sem_i)

    def wait_idx(j):
        pltpu.make_async_copy(src_h.at[pl.ds(0, ECHUNK)], srcs[j], sem_i).wait()
        pltpu.make_async_copy(tgt_h.at[pl.ds(0, ECHUNK)], tgts[j], sem_i).wait()

    def start_gathers(j, p):
        @pl.when(cid == 0)
        def _():
            pltpu.async_copy(q0.at[tgts[j]], qvs[p], sem_g)
            pltpu.async_copy(kv0.at[srcs[j]], kvs[p], sem_g)

        @pl.when(cid == 1)
        def _():
            pltpu.async_copy(q1.at[tgts[j]], qvs[p], sem_g)
            pltpu.async_copy(kv1.at[srcs[j]], kvs[p], sem_g)

    def wait_gathers(j, p):
        pltpu.make_async_copy(q0.at[tgts[j]], qvs[p], sem_g).wait()
        pltpu.make_async_copy(kv0.at[srcs[j]], kvs[p], sem_g).wait()

    def start_scatters(j, p):
        pltpu.async_copy(msgs[p], msg_acc.at[tgts[j]], sem_s, add=True)
        pltpu.async_copy(ex0s[p], den_acc0.at[tgts[j]], sem_s, add=True)
        pltpu.async_copy(ex1s[p], den_acc1.at[tgts[j]], sem_s, add=True)

    def wait_scatters(j, p):
        pltpu.make_async_copy(msgs[p], msg_acc.at[tgts[j]], sem_s).wait()
        pltpu.make_async_copy(ex0s[p], den_acc0.at[tgts[j]], sem_s).wait()
        pltpu.make_async_copy(ex1s[p], den_acc1.at[tgts[j]], sem_s).wait()

    def compute(p):
        q_v, kv_v = qvs[p], kvs[p]
        msg_v, ex0_v, ex1_v = msgs[p], ex0s[p], ex1s[p]

        def grp(i, c2):
            rows = i * LANES + iota
            acc0 = jnp.zeros((LANES,), jnp.float32)
            acc1 = jnp.zeros((LANES,), jnp.float32)
            for jj in range(HALF):
                col = jnp.full((LANES,), jj, jnp.int32)
                qq = plsc.load_gather(q_v, [rows, col])
                kk = plsc.load_gather(kv_v, [rows, col])
                if jj < C:
                    acc0 = acc0 + qq * kk
                else:
                    acc1 = acc1 + qq * kk
            ex0 = jnp.exp(acc0 * SCALE)
            ex1 = jnp.exp(acc1 * SCALE)
            ex0_v[pl.ds(i * LANES, LANES)] = ex0
            ex1_v[pl.ds(i * LANES, LANES)] = ex1
            for ch in range(HALF):
                col = jnp.full((LANES,), HALF + ch, jnp.int32)
                vv = plsc.load_gather(kv_v, [rows, col])
                m = vv * (ex0 if ch < C else ex1)
                plsc.store_scatter(msg_v, [rows, jnp.full((LANES,), ch, jnp.int32)], m)
            return c2

        lax.fori_loop(0, ECHUNK // LANES, grp, 0)

    start_idx(jnp.int32(0), 0)
    start_idx(jnp.int32(1), 1)
    wait_idx(0)
    start_gathers(0, 0)

    def six_body(t, carry):
        for b in range(6):
            g = t * 6 + b
            j = b % 3
            p = b % 2

            wait_gathers(j, p)

            @pl.when(g > 0)
            def _():
                wait_scatters((j + 2) % 3, 1 - p)
            start_idx(g + 2, (j + 2) % 3)
            wait_idx((j + 1) % 3)
            start_gathers((j + 1) % 3, 1 - p)
            compute(p)
            start_scatters(j, p)
        return carry

    lax.fori_loop(0, NCH_T // 6, six_body, 0)
    wait_gathers(0, 0)
    wait_idx(1)
    wait_scatters(2, 1)

    plsc.subcore_barrier()

    @pl.when(cid == 0)
    def _():
        pltpu.sync_copy(msg_acc.at[pl.ds(r0, ROWS_T)],
                        msg_out.at[0, pl.ds(r0, ROWS_T)])
        pltpu.sync_copy(den_acc0.at[pl.ds(r0, ROWS_T)],
                        den_out.at[0, 0, pl.ds(r0, ROWS_T)])
        pltpu.sync_copy(den_acc1.at[pl.ds(r0, ROWS_T)],
                        den_out.at[0, 1, pl.ds(r0, ROWS_T)])

    @pl.when(cid == 1)
    def _():
        pltpu.sync_copy(msg_acc.at[pl.ds(r0, ROWS_T)],
                        msg_out.at[1, pl.ds(r0, ROWS_T)])
        pltpu.sync_copy(den_acc0.at[pl.ds(r0, ROWS_T)],
                        den_out.at[1, 0, pl.ds(r0, ROWS_T)])
        pltpu.sync_copy(den_acc1.at[pl.ds(r0, ROWS_T)],
                        den_out.at[1, 1, pl.ds(r0, ROWS_T)])


def _edge_pass(q0, q1, kv0, kv1, src, tgt, zm, zd):
    f32 = jnp.float32
    kern = functools.partial(
        pl.kernel,
        out_type=(jax.ShapeDtypeStruct((NSC, N_PAD, HALF), f32),
                  jax.ShapeDtypeStruct((NSC, 2, N_PAD), f32)),
        mesh=plsc.VectorSubcoreMesh(core_axis_name="c", subcore_axis_name="s"),
        compiler_params=pltpu.CompilerParams(needs_layout_passes=False,
                                             use_tc_tiling_on_sc=False),
        scratch_types=(
            [pltpu.VMEM((ECHUNK,), jnp.int32)] * 6
            + [pltpu.VMEM((ECHUNK, HALF), f32)] * 2
            + [pltpu.VMEM((ECHUNK, D), f32)] * 2
            + [pltpu.VMEM((ECHUNK, HALF), f32)] * 2
            + [pltpu.VMEM((ECHUNK,), f32)] * 4
            + [
                pltpu.VMEM_SHARED((N_PAD, HALF), f32),
                pltpu.VMEM_SHARED((N_PAD,), f32),
                pltpu.VMEM_SHARED((N_PAD,), f32),
                pltpu.SemaphoreType.DMA,
                pltpu.SemaphoreType.DMA,
                pltpu.SemaphoreType.DMA,
            ]
        ),
    )(_edge_body)
    return kern(q0, q1, kv0, kv1, src, tgt, zm, zd)


def _combine_body(bc_ref, msg_ref, den_ref, ef_ref, skip_ref, wc_ref, s_ref):
    f32 = jnp.float32
    msg = jnp.concatenate([msg_ref[0], msg_ref[1]], axis=1)
    den4 = jnp.reshape(den_ref[...], (H, BLKA))
    eye = jax.lax.broadcasted_iota(jnp.int32, (H, H), 0)
    eye = (eye == jax.lax.broadcasted_iota(jnp.int32, (H, H), 1)).astype(f32)
    dn = (((0,), (0,)), ((), ()))
    den = lax.dot_general(den4, eye, dn, preferred_element_type=f32)
    invd = 1.0 / (den + 1e-16)
    sattn = den * invd
    ef = ef_ref[...]
    parts = []
    for h in range(H):
        parts.append(msg[:, h * C:(h + 1) * C] * invd[:, h:h + 1]
                     + ef[:, h * C:(h + 1) * C] * sattn[:, h:h + 1])
    x_trans = jnp.concatenate(parts, axis=1) + skip_ref[...]
    s_ref[...] = (jnp.dot(x_trans, wc_ref[...],
                          preferred_element_type=f32)
                  + 0.5 * bc_ref[0, 0])


def _combine(bc, msg, den, ef, skip, wc):
    return pl.pallas_call(
        _combine_body,
        grid=(N_PAD // BLKA,),
        in_specs=[
            pl.BlockSpec(memory_space=pltpu.SMEM),
            pl.BlockSpec((NSC, BLKA, HALF), lambda i: (0, i, 0)),
            pl.BlockSpec((NSC, 2, BLKA), lambda i: (0, 0, i)),
            pl.BlockSpec((BLKA, D), lambda i: (i, 0)),
            pl.BlockSpec((BLKA, D), lambda i: (i, 0)),
            pl.BlockSpec((D, 1), lambda i: (0, 0)),
        ],
        out_specs=pl.BlockSpec((BLKA, 1), lambda i: (i, 0)),
        out_shape=jax.ShapeDtypeStruct((N_PAD, 1), jnp.float32),
    )(bc, msg, den, ef, skip, wc)


def _edge_out_body(s_h, src_h, tgt_h, out_h, s_v, src_v, tgt_v, out_v):
    cid = lax.axis_index("c")
    sid = lax.axis_index("s")
    wid = sid * NSC + cid
    pltpu.sync_copy(s_h, s_v)

    def chunk_body(g, carry):
        base = wid * EPW + g * OCHUNK
        pltpu.sync_copy(src_h.at[pl.ds(base, OCHUNK)], src_v)
        pltpu.sync_copy(tgt_h.at[pl.ds(base, OCHUNK)], tgt_v)

        def grp(i, c2):
            o = i * LANES
            si = src_v[pl.ds(o, LANES)]
            ti = tgt_v[pl.ds(o, LANES)]
            sv = plsc.load_gather(s_v, [si])
            tv = plsc.load_gather(s_v, [ti])
            x = sv + tv
            out_v[pl.ds(o, LANES)] = 1.0 / (1.0 + jnp.exp(-x))
            return c2

        lax.fori_loop(0, OCHUNK // LANES, grp, 0)
        pltpu.sync_copy(out_v, out_h.at[pl.ds(base, OCHUNK)])
        return carry

    lax.fori_loop(0, EPW // OCHUNK, chunk_body, 0)


def _edge_out(s, src, tgt):
    return pl.kernel(
        _edge_out_body,
        out_type=jax.ShapeDtypeStruct((E,), jnp.float32),
        mesh=plsc.VectorSubcoreMesh(core_axis_name="c", subcore_axis_name="s"),
        compiler_params=pltpu.CompilerParams(needs_layout_passes=False),
        scratch_types=[
            pltpu.VMEM((N,), jnp.float32),
            pltpu.VMEM((OCHUNK,), jnp.int32),
            pltpu.VMEM((OCHUNK,), jnp.int32),
            pltpu.VMEM((OCHUNK,), jnp.float32),
        ],
    )(s, src, tgt)


def kernel(x_t, x_t_dt, edge_index, dt, W_node, b_node, W_edge, b_edge,
           W_q, b_q, W_k, b_k, W_v, b_v, W_skip, b_skip, W_cls, b_cls):
    f32 = jnp.float32
    dt_arr = jnp.reshape(jnp.asarray(dt, f32), (1, 1))
    mp = jnp.reshape(x_t[0, 1:4], (3, 1))
    xT = jnp.pad(jnp.transpose(x_t), ((0, 0), (0, N_PAD - N)))
    xdtT = jnp.pad(jnp.transpose(x_t_dt), ((0, 0), (0, N_PAD - N)))
    r = lambda b: jnp.reshape(b, (1, D))

    q0, q1, kv0, kv1, ef, skip = _node_precompute(
        dt_arr, mp, xT, xdtT, W_node, r(b_node), W_edge, r(b_edge),
        W_q, r(b_q), W_k, r(b_k), W_v, r(b_v), W_skip, r(b_skip))

    src = edge_index[0]
    tgt = edge_index[1]
    src_p = jnp.concatenate([src, jnp.zeros((E_PAD - E,), jnp.int32)])
    tgt_p = jnp.concatenate([tgt, jnp.full((E_PAD - E,), N_PAD - 1, jnp.int32)])
    zm = jnp.zeros((ROWS_T, HALF), f32)
    zd = jnp.zeros((ROWS_T,), f32)
    msg, den = _edge_pass(q0, q1, kv0, kv1, src_p, tgt_p, zm, zd)

    bc = jnp.reshape(b_cls, (1, 1))
    s = _combine(bc, msg, den, ef, skip, W_cls)

    return _edge_out(jnp.reshape(s, (N_PAD,))[:N], src, tgt)
